# baseline (speedup 1.0000x reference)
import jax
import jax.numpy as jnp
from jax import lax
from jax.experimental import pallas as pl
from jax.experimental.pallas import tpu as pltpu
from jax.experimental.pallas import tpu_sc as plsc

_N = 10000
_NPAD = 10240
_E = 320000
_B = 128
_TILES = 16
_W = 32
_EPT = 20224
_NCH = _EPT // _B
_HALF = _NPAD // 2
_ACC_R = _HALF + 8
_TRASH = _HALF
_RPT = _HALF // _TILES


def _make_prop():
    mesh = plsc.VectorSubcoreMesh(core_axis_name="c", subcore_axis_name="s")

    def body(t, src_r, dst_r, out, src_v, dst_v, rows0, rows1, acc,
             gsem0, gsem1):
        c = lax.axis_index("c")
        s = lax.axis_index("s")
        pltpu.sync_copy(src_r.at[s], src_v)
        pltpu.sync_copy(dst_r.at[s], dst_v)

        nbase = c * _HALF

        def remap(g, carry):
            for j in range(_B // 16):
                v = dst_v[g, pl.ds(j * 16, 16)] - nbase
                ok = (v >= 0) & (v < _HALF)
                dst_v[g, pl.ds(j * 16, 16)] = jnp.where(ok, v, _TRASH)
            return carry

        lax.fori_loop(0, _NCH, remap, 0)

        zero = jnp.zeros((16,), jnp.float32)

        def zrow(i, carry):
            for j in range(128 // 16):
                rows0[i, pl.ds(j * 16, 16)] = zero
            return carry

        lax.fori_loop(0, _B, zrow, 0)
        base = s * _RPT
        pltpu.sync_copy(rows0, acc.at[pl.ds(base, 128)])
        pltpu.sync_copy(rows0, acc.at[pl.ds(base + 128, 128)])
        pltpu.sync_copy(rows0.at[pl.ds(0, 64)], acc.at[pl.ds(base + 256, 64)])

        @pl.when(s == 0)
        def _():
            pltpu.sync_copy(rows0.at[pl.ds(0, 8)], acc.at[pl.ds(_HALF, 8)])

        plsc.subcore_barrier()

        def step(g, carry):
            pltpu.async_copy(t.at[src_v.at[g]], rows0, gsem0).wait()
            pltpu.sync_copy(rows0, acc.at[dst_v.at[g]], add=True)
            return carry

        lax.fori_loop(0, _NCH, step, 0)
        plsc.subcore_barrier()
        pltpu.sync_copy(acc.at[pl.ds(base, _RPT)],
                        out.at[pl.ds(nbase + base, _RPT)])

    return pl.kernel(
        body,
        out_type=jax.ShapeDtypeStruct((_NPAD, 128), jnp.float32),
        mesh=mesh,
        scratch_types=[
            pltpu.VMEM((_NCH, _B), jnp.int32),
            pltpu.VMEM((_NCH, _B), jnp.int32),
            pltpu.VMEM((_B, 128), jnp.float32),
            pltpu.VMEM((_B, 128), jnp.float32),
            pltpu.VMEM_SHARED((_ACC_R, 128), jnp.float32),
            pltpu.SemaphoreType.DMA,
            pltpu.SemaphoreType.DMA,
        ],
    )


def _make_deg():
    mesh = plsc.VectorSubcoreMesh(core_axis_name="c", subcore_axis_name="s")
    epw = _E // _W

    def body(dst_r, out, dst_v, hist):
        c = lax.axis_index("c")
        s = lax.axis_index("s")
        wid = c * _TILES + s
        pltpu.sync_copy(dst_r.at[wid], dst_v)

        zero = jnp.zeros((16,), jnp.float32)

        def zrow(i, carry):
            hist[pl.ds(i * 16, 16)] = zero
            return carry

        lax.fori_loop(0, _NPAD // 16, zrow, 0)

        one = jnp.ones((16,), jnp.float32)

        def step(i, carry):
            idx = dst_v[pl.ds(i * 16, 16)]
            plsc.addupdate_scatter(hist, [idx], one)
            return carry

        lax.fori_loop(0, epw // 16, step, 0)
        pltpu.sync_copy(hist, out.at[wid])

    return pl.kernel(
        body,
        out_type=jax.ShapeDtypeStruct((_W, _NPAD), jnp.float32),
        mesh=mesh,
        compiler_params=pltpu.CompilerParams(needs_layout_passes=False),
        scratch_types=[
            pltpu.VMEM((epw,), jnp.int32),
            pltpu.VMEM((_NPAD,), jnp.float32),
        ],
    )


_PROP = _make_prop()
_DEG = _make_deg()

_RB = 1000
_GRID = _N // _RB


def _tc_a(x, hists):
    def body(x_r, h_r, t0_r, dv_r):
        d = jnp.sum(h_r[...], axis=1)
        dinv_row = jnp.where(d > 0.0, lax.rsqrt(jnp.maximum(d, 1.0)), 0.0)
        dinv = jnp.transpose(dinv_row)
        t0_r[...] = x_r[...] * dinv
        dv_r[...] = dinv

    return pl.pallas_call(
        body,
        grid=(_GRID,),
        in_specs=[
            pl.BlockSpec((_RB, 128), lambda i: (i, 0)),
            pl.BlockSpec((1, _W, _RB), lambda i: (i, 0, 0)),
        ],
        out_specs=[
            pl.BlockSpec((_RB, 128), lambda i: (i, 0)),
            pl.BlockSpec((_RB, 1), lambda i: (i, 0)),
        ],
        out_shape=[
            jax.ShapeDtypeStruct((_N, 128), jnp.float32),
            jax.ShapeDtypeStruct((_N, 1), jnp.float32),
        ],
    )(x, hists)


def _tc_b(p, dinv, w1a, b1a, w1b, b1b):
    def body(p_r, dv_r, wa_r, ba_r, wb_r, bb_r, t2a_r, t2b_r):
        dv = dv_r[...]
        px = p_r[...] * dv
        ha = jnp.maximum(
            jnp.dot(px, wa_r[...], preferred_element_type=jnp.float32)
            + ba_r[...], 0.0)
        hb = jnp.maximum(
            jnp.dot(px, wb_r[...], preferred_element_type=jnp.float32)
            + bb_r[...], 0.0)
        t2a_r[...] = ha * dv
        t2b_r[...] = hb * dv

    return pl.pallas_call(
        body,
        grid=(_GRID,),
        in_specs=[
            pl.BlockSpec((_RB, 128), lambda i: (i, 0)),
            pl.BlockSpec((_RB, 1), lambda i: (i, 0)),
            pl.BlockSpec((128, 128), lambda i: (0, 0)),
            pl.BlockSpec((1, 128), lambda i: (0, 0)),
            pl.BlockSpec((128, 128), lambda i: (0, 0)),
            pl.BlockSpec((1, 128), lambda i: (0, 0)),
        ],
        out_specs=[
            pl.BlockSpec((_RB, 128), lambda i: (i, 0)),
            pl.BlockSpec((_RB, 128), lambda i: (i, 0)),
        ],
        out_shape=[
            jax.ShapeDtypeStruct((_N, 128), jnp.float32),
            jax.ShapeDtypeStruct((_N, 128), jnp.float32),
        ],
    )(p, dinv, w1a, b1a, w1b, b1b)


def _tc_c(qa, qb, dinv, w2a, b2a, w2b, b2b):
    def body(qa_r, qb_r, dv_r, wa_r, ba_r, wb_r, bb_r, o1_r, o2_r):
        dv = dv_r[...]
        for q_r, w_r, b_r, o_r in ((qa_r, wa_r, ba_r, o1_r),
                                   (qb_r, wb_r, bb_r, o2_r)):
            s2 = q_r[...] * dv
            z = jnp.dot(s2, w_r[...], preferred_element_type=jnp.float32) \
                + b_r[...]
            m = jnp.max(z, axis=-1, keepdims=True)
            lse = jnp.log(jnp.sum(jnp.exp(z - m), axis=-1, keepdims=True)) + m
            o_r[...] = z - lse

    return pl.pallas_call(
        body,
        grid=(_GRID,),
        in_specs=[
            pl.BlockSpec((_RB, 128), lambda i: (i, 0)),
            pl.BlockSpec((_RB, 128), lambda i: (i, 0)),
            pl.BlockSpec((_RB, 1), lambda i: (i, 0)),
            pl.BlockSpec((128, 64), lambda i: (0, 0)),
            pl.BlockSpec((1, 64), lambda i: (0, 0)),
            pl.BlockSpec((128, 64), lambda i: (0, 0)),
            pl.BlockSpec((1, 64), lambda i: (0, 0)),
        ],
        out_specs=[
            pl.BlockSpec((_RB, 64), lambda i: (i, 0)),
            pl.BlockSpec((_RB, 64), lambda i: (i, 0)),
        ],
        out_shape=[
            jax.ShapeDtypeStruct((_N, 64), jnp.float32),
            jax.ShapeDtypeStruct((_N, 64), jnp.float32),
        ],
    )(qa, qb, dinv, w2a, b2a, w2b, b2b)


def kernel(x, edge_index, W1a, b1a, W2a, b2a, W1b, b1b, W2b, b2b):
    ept = _E // _TILES
    pad = _EPT - ept
    src = jnp.concatenate(
        [edge_index[0].reshape(_TILES, ept),
         jnp.zeros((_TILES, pad), jnp.int32)], axis=1
    ).reshape(_TILES, _NCH, _B)
    dst = jnp.concatenate(
        [edge_index[1].reshape(_TILES, ept),
         jnp.full((_TILES, pad), -1, jnp.int32)], axis=1
    ).reshape(_TILES, _NCH, _B)
    dst_flat = edge_index[1].reshape(_W, _E // _W)

    hists = _DEG(dst_flat)
    hists_t = jnp.transpose(hists[:, :_N].reshape(_W, _GRID, _RB), (1, 0, 2))
    t0, dinv = _tc_a(x, hists_t)
    p = _PROP(t0, src, dst)
    t2a, t2b = _tc_b(p[:_N], dinv, W1a, b1a.reshape(1, 128),
                     W1b, b1b.reshape(1, 128))
    qa = _PROP(t2a, src, dst)
    qb = _PROP(t2b, src, dst)
    return _tc_c(qa[:_N], qb[:_N], dinv, W2a, b2a.reshape(1, 64),
                 W2b, b2b.reshape(1, 64))


# speedup vs baseline: 2.1091x; 2.1091x over previous
import jax
import jax.numpy as jnp
from jax import lax
from jax.experimental import pallas as pl
from jax.experimental.pallas import tpu as pltpu
from jax.experimental.pallas import tpu_sc as plsc

_N = 10000
_NPAD = 10240
_E = 320000
_B = 80
_TILES = 16
_W = 32
_EPT = _E // _TILES
_BLK = 2000
_NBLK = _EPT // _BLK
_SEL = 20160
_HALF = _NPAD // 2
_ACC_R = _HALF + 8
_TRASH = _HALF
_RPT = _HALF // _TILES


def _make_prop():
    mesh = plsc.VectorSubcoreMesh(core_axis_name="c", subcore_axis_name="s")

    def body(t, src_r, dst_r, out, srcb, dstb, src_sel, dst_self, dst_sel2,
             rows0, acc, gsem0):
        c = lax.axis_index("c")
        s = lax.axis_index("s")

        nbase = c * _HALF

        def vec(i, cur):
            vsrc = srcb[pl.ds(16 * i, 16)]
            vdst = dstb[pl.ds(16 * i, 16)] - nbase
            ok = (vdst >= 0) & (vdst < _HALF)
            plsc.store_compressed(src_sel.at[pl.ds(cur, 16)], vsrc, mask=ok)
            plsc.store_compressed(dst_self.at[pl.ds(cur, 16)], vdst, mask=ok)
            return cur + jnp.sum(jnp.where(ok, 1, 0))

        def block(b, cur):
            pltpu.sync_copy(src_r.at[s * _NBLK + b], srcb)
            pltpu.sync_copy(dst_r.at[s * _NBLK + b], dstb)
            return lax.fori_loop(0, _BLK // 16, vec, cur)

        cursor = lax.fori_loop(0, _NBLK, block, 0)

        zero_i = jnp.zeros((16,), jnp.int32)
        trash = jnp.full((16,), _TRASH, jnp.int32)
        for j in range(_B // 16):
            src_sel[pl.ds(cursor + 16 * j, 16)] = zero_i
            dst_self[pl.ds(cursor + 16 * j, 16)] = trash
        nch = (cursor + _B - 1) // _B

        def crow(g, carry):
            for j in range(_B // 16):
                dst_sel2[g, pl.ds(16 * j, 16)] = \
                    dst_self[pl.ds(_B * g + 16 * j, 16)]
            return carry

        lax.fori_loop(0, nch, crow, 0)

        zero = jnp.zeros((16,), jnp.float32)

        def zrow(i, carry):
            for j in range(128 // 16):
                rows0[i, pl.ds(j * 16, 16)] = zero
            return carry

        lax.fori_loop(0, _B, zrow, 0)
        base = s * _RPT
        for j in range(_RPT // _B):
            pltpu.sync_copy(rows0, acc.at[pl.ds(base + j * _B, _B)])

        @pl.when(s == 0)
        def _():
            pltpu.sync_copy(rows0.at[pl.ds(0, 8)], acc.at[pl.ds(_HALF, 8)])

        plsc.subcore_barrier()

        def step(g, carry):
            idx = src_sel.at[pl.ds(g * _B, _B)]
            pltpu.async_copy(t.at[idx], rows0, gsem0).wait()
            pltpu.sync_copy(rows0, acc.at[dst_sel2.at[g]], add=True)
            return carry

        lax.fori_loop(0, nch, step, 0)
        plsc.subcore_barrier()
        pltpu.sync_copy(acc.at[pl.ds(base, _RPT)],
                        out.at[pl.ds(nbase + base, _RPT)])

    return pl.kernel(
        body,
        out_type=jax.ShapeDtypeStruct((_NPAD, 128), jnp.float32),
        mesh=mesh,
        compiler_params=pltpu.CompilerParams(needs_layout_passes=False),
        scratch_types=[
            pltpu.VMEM((_BLK,), jnp.int32),
            pltpu.VMEM((_BLK,), jnp.int32),
            pltpu.VMEM((_SEL,), jnp.int32),
            pltpu.VMEM((_SEL,), jnp.int32),
            pltpu.VMEM((_SEL // _B, _B), jnp.int32),
            pltpu.VMEM((_B, 128), jnp.float32),
            pltpu.VMEM_SHARED((_ACC_R, 128), jnp.float32),
            pltpu.SemaphoreType.DMA,
        ],
    )


def _make_deg():
    mesh = plsc.VectorSubcoreMesh(core_axis_name="c", subcore_axis_name="s")
    epw = _E // _W

    def body(dst_r, out, dst_v, hist):
        c = lax.axis_index("c")
        s = lax.axis_index("s")
        wid = c * _TILES + s
        pltpu.sync_copy(dst_r.at[wid], dst_v)

        zero = jnp.zeros((16,), jnp.float32)

        def zrow(i, carry):
            hist[pl.ds(i * 16, 16)] = zero
            return carry

        lax.fori_loop(0, _NPAD // 16, zrow, 0)

        one = jnp.ones((16,), jnp.float32)

        def step(i, carry):
            idx = dst_v[pl.ds(i * 16, 16)]
            plsc.addupdate_scatter(hist, [idx], one)
            return carry

        lax.fori_loop(0, epw // 16, step, 0)
        pltpu.sync_copy(hist, out.at[wid])

    return pl.kernel(
        body,
        out_type=jax.ShapeDtypeStruct((_W, _NPAD), jnp.float32),
        mesh=mesh,
        compiler_params=pltpu.CompilerParams(needs_layout_passes=False),
        scratch_types=[
            pltpu.VMEM((epw,), jnp.int32),
            pltpu.VMEM((_NPAD,), jnp.float32),
        ],
    )


_PROP = _make_prop()
_DEG = _make_deg()

_RB = 1000
_GRID = _N // _RB


def _tc_a(x, hists):
    def body(x_r, h_r, t0_r, dv_r):
        d = jnp.sum(h_r[...], axis=1)
        dinv_row = jnp.where(d > 0.0, lax.rsqrt(jnp.maximum(d, 1.0)), 0.0)
        dinv = jnp.transpose(dinv_row)
        t0_r[...] = x_r[...] * dinv
        dv_r[...] = dinv

    return pl.pallas_call(
        body,
        grid=(_GRID,),
        in_specs=[
            pl.BlockSpec((_RB, 128), lambda i: (i, 0)),
            pl.BlockSpec((1, _W, _RB), lambda i: (i, 0, 0)),
        ],
        out_specs=[
            pl.BlockSpec((_RB, 128), lambda i: (i, 0)),
            pl.BlockSpec((_RB, 1), lambda i: (i, 0)),
        ],
        out_shape=[
            jax.ShapeDtypeStruct((_N, 128), jnp.float32),
            jax.ShapeDtypeStruct((_N, 1), jnp.float32),
        ],
    )(x, hists)


def _tc_b(p, dinv, w1a, b1a, w1b, b1b):
    def body(p_r, dv_r, wa_r, ba_r, wb_r, bb_r, t2a_r, t2b_r):
        dv = dv_r[...]
        px = p_r[...] * dv
        ha = jnp.maximum(
            jnp.dot(px, wa_r[...], preferred_element_type=jnp.float32)
            + ba_r[...], 0.0)
        hb = jnp.maximum(
            jnp.dot(px, wb_r[...], preferred_element_type=jnp.float32)
            + bb_r[...], 0.0)
        t2a_r[...] = ha * dv
        t2b_r[...] = hb * dv

    return pl.pallas_call(
        body,
        grid=(_GRID,),
        in_specs=[
            pl.BlockSpec((_RB, 128), lambda i: (i, 0)),
            pl.BlockSpec((_RB, 1), lambda i: (i, 0)),
            pl.BlockSpec((128, 128), lambda i: (0, 0)),
            pl.BlockSpec((1, 128), lambda i: (0, 0)),
            pl.BlockSpec((128, 128), lambda i: (0, 0)),
            pl.BlockSpec((1, 128), lambda i: (0, 0)),
        ],
        out_specs=[
            pl.BlockSpec((_RB, 128), lambda i: (i, 0)),
            pl.BlockSpec((_RB, 128), lambda i: (i, 0)),
        ],
        out_shape=[
            jax.ShapeDtypeStruct((_N, 128), jnp.float32),
            jax.ShapeDtypeStruct((_N, 128), jnp.float32),
        ],
    )(p, dinv, w1a, b1a, w1b, b1b)


def _tc_c(qa, qb, dinv, w2a, b2a, w2b, b2b):
    def body(qa_r, qb_r, dv_r, wa_r, ba_r, wb_r, bb_r, o1_r, o2_r):
        dv = dv_r[...]
        for q_r, w_r, b_r, o_r in ((qa_r, wa_r, ba_r, o1_r),
                                   (qb_r, wb_r, bb_r, o2_r)):
            s2 = q_r[...] * dv
            z = jnp.dot(s2, w_r[...], preferred_element_type=jnp.float32) \
                + b_r[...]
            m = jnp.max(z, axis=-1, keepdims=True)
            lse = jnp.log(jnp.sum(jnp.exp(z - m), axis=-1, keepdims=True)) + m
            o_r[...] = z - lse

    return pl.pallas_call(
        body,
        grid=(_GRID,),
        in_specs=[
            pl.BlockSpec((_RB, 128), lambda i: (i, 0)),
            pl.BlockSpec((_RB, 128), lambda i: (i, 0)),
            pl.BlockSpec((_RB, 1), lambda i: (i, 0)),
            pl.BlockSpec((128, 64), lambda i: (0, 0)),
            pl.BlockSpec((1, 64), lambda i: (0, 0)),
            pl.BlockSpec((128, 64), lambda i: (0, 0)),
            pl.BlockSpec((1, 64), lambda i: (0, 0)),
        ],
        out_specs=[
            pl.BlockSpec((_RB, 64), lambda i: (i, 0)),
            pl.BlockSpec((_RB, 64), lambda i: (i, 0)),
        ],
        out_shape=[
            jax.ShapeDtypeStruct((_N, 64), jnp.float32),
            jax.ShapeDtypeStruct((_N, 64), jnp.float32),
        ],
    )(qa, qb, dinv, w2a, b2a, w2b, b2b)


def kernel(x, edge_index, W1a, b1a, W2a, b2a, W1b, b1b, W2b, b2b):
    src = edge_index[0].reshape(_TILES * _NBLK, _BLK)
    dst = edge_index[1].reshape(_TILES * _NBLK, _BLK)
    dst_flat = edge_index[1].reshape(_W, _E // _W)

    hists = _DEG(dst_flat)
    hists_t = jnp.transpose(hists[:, :_N].reshape(_W, _GRID, _RB), (1, 0, 2))
    t0, dinv = _tc_a(x, hists_t)
    p = _PROP(t0, src, dst)
    t2a, t2b = _tc_b(p[:_N], dinv, W1a, b1a.reshape(1, 128),
                     W1b, b1b.reshape(1, 128))
    qa = _PROP(t2a, src, dst)
    qb = _PROP(t2b, src, dst)
    return _tc_c(qa[:_N], qb[:_N], dinv, W2a, b2a.reshape(1, 64),
                 W2b, b2b.reshape(1, 64))


# speedup vs baseline: 2.9493x; 1.3984x over previous
import jax
import jax.numpy as jnp
from jax import lax
from jax.experimental import pallas as pl
from jax.experimental.pallas import tpu as pltpu
from jax.experimental.pallas import tpu_sc as plsc

_N = 10000
_NPAD = 10240
_E = 320000
_B = 80
_TILES = 16
_W = 32
_EPT = _E // _TILES
_BLK = 2000
_NBLK = _EPT // _BLK
_SEL = 20160
_HALF = _NPAD // 2
_ACC_R = _HALF + 8
_TRASH = _HALF
_RPT = _HALF // _TILES


def _make_prop():
    mesh = plsc.VectorSubcoreMesh(core_axis_name="c", subcore_axis_name="s")

    def body(t, src_r, dst_r, out, srcb, dstb, src_sel, dst_self,
             rows0, rows1, acc, gsem0, gsem1):
        c = lax.axis_index("c")
        s = lax.axis_index("s")

        nbase = c * _HALF

        def vec(i, cur):
            vsrc = srcb[pl.ds(16 * i, 16)]
            vdst = dstb[pl.ds(16 * i, 16)] - nbase
            ok = (vdst >= 0) & (vdst < _HALF)
            plsc.store_compressed(src_sel.at[pl.ds(cur, 16)], vsrc, mask=ok)
            plsc.store_compressed(dst_self.at[pl.ds(cur, 16)], vdst, mask=ok)
            return cur + jnp.sum(jnp.where(ok, 1, 0))

        def block(b, cur):
            pltpu.sync_copy(src_r.at[s * _NBLK + b], srcb)
            pltpu.sync_copy(dst_r.at[s * _NBLK + b], dstb)
            return lax.fori_loop(0, _BLK // 16, vec, cur)

        cursor = lax.fori_loop(0, _NBLK, block, 0)

        zero_i = jnp.zeros((16,), jnp.int32)
        trash = jnp.full((16,), _TRASH, jnp.int32)
        for j in range(_B // 16):
            src_sel[pl.ds(cursor + 16 * j, 16)] = zero_i
            dst_self[pl.ds(cursor + 16 * j, 16)] = trash
        nch = (cursor + _B - 1) // _B

        zero = jnp.zeros((16,), jnp.float32)

        def zrow(i, carry):
            for j in range(128 // 16):
                rows0[i, pl.ds(j * 16, 16)] = zero
            return carry

        lax.fori_loop(0, _B, zrow, 0)
        base = s * _RPT
        for j in range(_RPT // _B):
            pltpu.sync_copy(rows0, acc.at[pl.ds(base + j * _B, _B)])

        @pl.when(s == 0)
        def _():
            pltpu.sync_copy(rows0.at[pl.ds(0, 8)], acc.at[pl.ds(_HALF, 8)])

        plsc.subcore_barrier()

        def gat(g, buf, sem):
            pltpu.async_copy(t.at[src_sel.at[pl.ds(g * _B, _B)]], buf, sem)

        def sca(g, buf):
            pltpu.sync_copy(buf, acc.at[dst_self.at[pl.ds(g * _B, _B)]],
                            add=True)

        @pl.when(nch > 0)
        def _():
            gat(0, rows0, gsem0)

        def pair(k, carry):
            g = 2 * k
            gat(g + 1, rows1, gsem1)
            pltpu.make_async_copy(t.at[src_sel.at[pl.ds(0, _B)]], rows0,
                                  gsem0).wait()
            sca(g, rows0)

            @pl.when(g + 2 < nch)
            def _():
                gat(g + 2, rows0, gsem0)

            pltpu.make_async_copy(t.at[src_sel.at[pl.ds(0, _B)]], rows1,
                                  gsem1).wait()
            sca(g + 1, rows1)
            return carry

        lax.fori_loop(0, nch // 2, pair, 0)

        @pl.when(nch % 2 == 1)
        def _():
            pltpu.make_async_copy(t.at[src_sel.at[pl.ds(0, _B)]], rows0,
                                  gsem0).wait()
            sca(nch - 1, rows0)

        plsc.subcore_barrier()
        pltpu.sync_copy(acc.at[pl.ds(base, _RPT)],
                        out.at[pl.ds(nbase + base, _RPT)])

    return pl.kernel(
        body,
        out_type=jax.ShapeDtypeStruct((_NPAD, 128), jnp.float32),
        mesh=mesh,
        compiler_params=pltpu.CompilerParams(needs_layout_passes=False),
        scratch_types=[
            pltpu.VMEM((_BLK,), jnp.int32),
            pltpu.VMEM((_BLK,), jnp.int32),
            pltpu.VMEM((_SEL,), jnp.int32),
            pltpu.VMEM((_SEL,), jnp.int32),
            pltpu.VMEM((_B, 128), jnp.float32),
            pltpu.VMEM((_B, 128), jnp.float32),
            pltpu.VMEM_SHARED((_ACC_R, 128), jnp.float32),
            pltpu.SemaphoreType.DMA,
            pltpu.SemaphoreType.DMA,
        ],
    )


def _make_deg():
    mesh = plsc.VectorSubcoreMesh(core_axis_name="c", subcore_axis_name="s")
    epw = _E // _W

    def body(dst_r, out, dst_v, hist):
        c = lax.axis_index("c")
        s = lax.axis_index("s")
        wid = c * _TILES + s
        pltpu.sync_copy(dst_r.at[wid], dst_v)

        zero = jnp.zeros((16,), jnp.float32)

        def zrow(i, carry):
            hist[pl.ds(i * 16, 16)] = zero
            return carry

        lax.fori_loop(0, _NPAD // 16, zrow, 0)

        one = jnp.ones((16,), jnp.float32)

        def step(i, carry):
            idx = dst_v[pl.ds(i * 16, 16)]
            plsc.addupdate_scatter(hist, [idx], one)
            return carry

        lax.fori_loop(0, epw // 16, step, 0)
        pltpu.sync_copy(hist, out.at[wid])

    return pl.kernel(
        body,
        out_type=jax.ShapeDtypeStruct((_W, _NPAD), jnp.float32),
        mesh=mesh,
        compiler_params=pltpu.CompilerParams(needs_layout_passes=False),
        scratch_types=[
            pltpu.VMEM((epw,), jnp.int32),
            pltpu.VMEM((_NPAD,), jnp.float32),
        ],
    )


_PROP = _make_prop()
_DEG = _make_deg()

_RB = 1000
_GRID = _N // _RB


def _tc_a(x, hists):
    def body(x_r, h_r, t0_r, dv_r):
        d = jnp.sum(h_r[...], axis=1)
        dinv_row = jnp.where(d > 0.0, lax.rsqrt(jnp.maximum(d, 1.0)), 0.0)
        dinv = jnp.transpose(dinv_row)
        t0_r[...] = x_r[...] * dinv
        dv_r[...] = dinv

    return pl.pallas_call(
        body,
        grid=(_GRID,),
        in_specs=[
            pl.BlockSpec((_RB, 128), lambda i: (i, 0)),
            pl.BlockSpec((1, _W, _RB), lambda i: (i, 0, 0)),
        ],
        out_specs=[
            pl.BlockSpec((_RB, 128), lambda i: (i, 0)),
            pl.BlockSpec((_RB, 1), lambda i: (i, 0)),
        ],
        out_shape=[
            jax.ShapeDtypeStruct((_N, 128), jnp.float32),
            jax.ShapeDtypeStruct((_N, 1), jnp.float32),
        ],
    )(x, hists)


def _tc_b(p, dinv, w1a, b1a, w1b, b1b):
    def body(p_r, dv_r, wa_r, ba_r, wb_r, bb_r, t2a_r, t2b_r):
        dv = dv_r[...]
        px = p_r[...] * dv
        ha = jnp.maximum(
            jnp.dot(px, wa_r[...], preferred_element_type=jnp.float32)
            + ba_r[...], 0.0)
        hb = jnp.maximum(
            jnp.dot(px, wb_r[...], preferred_element_type=jnp.float32)
            + bb_r[...], 0.0)
        t2a_r[...] = ha * dv
        t2b_r[...] = hb * dv

    return pl.pallas_call(
        body,
        grid=(_GRID,),
        in_specs=[
            pl.BlockSpec((_RB, 128), lambda i: (i, 0)),
            pl.BlockSpec((_RB, 1), lambda i: (i, 0)),
            pl.BlockSpec((128, 128), lambda i: (0, 0)),
            pl.BlockSpec((1, 128), lambda i: (0, 0)),
            pl.BlockSpec((128, 128), lambda i: (0, 0)),
            pl.BlockSpec((1, 128), lambda i: (0, 0)),
        ],
        out_specs=[
            pl.BlockSpec((_RB, 128), lambda i: (i, 0)),
            pl.BlockSpec((_RB, 128), lambda i: (i, 0)),
        ],
        out_shape=[
            jax.ShapeDtypeStruct((_N, 128), jnp.float32),
            jax.ShapeDtypeStruct((_N, 128), jnp.float32),
        ],
    )(p, dinv, w1a, b1a, w1b, b1b)


def _tc_c(qa, qb, dinv, w2a, b2a, w2b, b2b):
    def body(qa_r, qb_r, dv_r, wa_r, ba_r, wb_r, bb_r, o1_r, o2_r):
        dv = dv_r[...]
        for q_r, w_r, b_r, o_r in ((qa_r, wa_r, ba_r, o1_r),
                                   (qb_r, wb_r, bb_r, o2_r)):
            s2 = q_r[...] * dv
            z = jnp.dot(s2, w_r[...], preferred_element_type=jnp.float32) \
                + b_r[...]
            m = jnp.max(z, axis=-1, keepdims=True)
            lse = jnp.log(jnp.sum(jnp.exp(z - m), axis=-1, keepdims=True)) + m
            o_r[...] = z - lse

    return pl.pallas_call(
        body,
        grid=(_GRID,),
        in_specs=[
            pl.BlockSpec((_RB, 128), lambda i: (i, 0)),
            pl.BlockSpec((_RB, 128), lambda i: (i, 0)),
            pl.BlockSpec((_RB, 1), lambda i: (i, 0)),
            pl.BlockSpec((128, 64), lambda i: (0, 0)),
            pl.BlockSpec((1, 64), lambda i: (0, 0)),
            pl.BlockSpec((128, 64), lambda i: (0, 0)),
            pl.BlockSpec((1, 64), lambda i: (0, 0)),
        ],
        out_specs=[
            pl.BlockSpec((_RB, 64), lambda i: (i, 0)),
            pl.BlockSpec((_RB, 64), lambda i: (i, 0)),
        ],
        out_shape=[
            jax.ShapeDtypeStruct((_N, 64), jnp.float32),
            jax.ShapeDtypeStruct((_N, 64), jnp.float32),
        ],
    )(qa, qb, dinv, w2a, b2a, w2b, b2b)


def kernel(x, edge_index, W1a, b1a, W2a, b2a, W1b, b1b, W2b, b2b):
    src = edge_index[0].reshape(_TILES * _NBLK, _BLK)
    dst = edge_index[1].reshape(_TILES * _NBLK, _BLK)
    dst_flat = edge_index[1].reshape(_W, _E // _W)

    hists = _DEG(dst_flat)
    hists_t = jnp.transpose(hists[:, :_N].reshape(_W, _GRID, _RB), (1, 0, 2))
    t0, dinv = _tc_a(x, hists_t)
    p = _PROP(t0, src, dst)
    t2a, t2b = _tc_b(p[:_N], dinv, W1a, b1a.reshape(1, 128),
                     W1b, b1b.reshape(1, 128))
    qa = _PROP(t2a, src, dst)
    qb = _PROP(t2b, src, dst)
    return _tc_c(qa[:_N], qb[:_N], dinv, W2a, b2a.reshape(1, 64),
                 W2b, b2b.reshape(1, 64))
